# baseline (speedup 1.0000x reference)
import functools

import jax
import jax.numpy as jnp
from jax import lax
from jax.experimental import pallas as pl
from jax.experimental.pallas import tpu as pltpu
from jax.experimental.pallas import tpu_sc as plsc

N_NODES = 10000
N_PAD = 10112
E_EDGES = 320000
D = 128
NW = 32
CHUNK = 128
CPW = 80
E_PAD = NW * CPW * CHUNK
ROWS_PER_TILE = N_PAD // 16
B_SEL = 4096
SEL_PER_W = B_SEL // NW

_sc_mesh = plsc.VectorSubcoreMesh(core_axis_name="c", subcore_axis_name="s")


@functools.partial(
    pl.kernel,
    out_type=jax.ShapeDtypeStruct((2, N_PAD, D), jnp.float32),
    mesh=_sc_mesh,
    scratch_types=[
        pltpu.VMEM((CPW, CHUNK), jnp.int32),
        pltpu.VMEM((CHUNK, D), jnp.float32),
        pltpu.VMEM_SHARED((N_PAD, D), jnp.float32),
    ],
)
def _deg_kernel(dst2_hbm, ones_hbm, zeros_hbm, deg_out, dst_v, ones_v, acc_sh):
    c = lax.axis_index("c")
    s = lax.axis_index("s")
    pltpu.sync_copy(zeros_hbm.at[pl.ds(s * ROWS_PER_TILE, ROWS_PER_TILE)],
                    acc_sh.at[pl.ds(s * ROWS_PER_TILE, ROWS_PER_TILE)])
    pltpu.sync_copy(ones_hbm, ones_v)
    wid = s * 2 + c
    pltpu.sync_copy(dst2_hbm.at[pl.ds(wid * CPW, CPW)], dst_v)
    plsc.subcore_barrier()

    def body(j, carry):
        pltpu.sync_copy(ones_v, acc_sh.at[dst_v.at[j]], add=True)
        return carry

    lax.fori_loop(0, CPW, body, 0)
    plsc.subcore_barrier()
    pltpu.sync_copy(acc_sh.at[pl.ds(s * ROWS_PER_TILE, ROWS_PER_TILE)],
                    deg_out.at[c, pl.ds(s * ROWS_PER_TILE, ROWS_PER_TILE)])


@functools.partial(
    pl.kernel,
    out_type=jax.ShapeDtypeStruct((2, N_PAD, D), jnp.float32),
    mesh=_sc_mesh,
    scratch_types=[
        pltpu.VMEM((CPW, CHUNK), jnp.int32),
        pltpu.VMEM((1, CHUNK), jnp.int32),
        pltpu.VMEM((1, CHUNK), jnp.int32),
        pltpu.VMEM((CHUNK, D), jnp.float32),
        pltpu.VMEM((CHUNK, D), jnp.float32),
        pltpu.SemaphoreType.DMA,
        pltpu.SemaphoreType.DMA,
        pltpu.SemaphoreType.DMA,
        pltpu.SemaphoreType.DMA,
        pltpu.VMEM_SHARED((N_PAD, D), jnp.float32),
    ],
)
def _seg_kernel(g_hbm, src2_hbm, dst2_hbm, zeros_hbm, acc_out, dst_v, s0, s1,
                rows0, rows1, gsem0, gsem1, ssem0, ssem1, acc_sh):
    c = lax.axis_index("c")
    s = lax.axis_index("s")
    pltpu.sync_copy(zeros_hbm.at[pl.ds(s * ROWS_PER_TILE, ROWS_PER_TILE)],
                    acc_sh.at[pl.ds(s * ROWS_PER_TILE, ROWS_PER_TILE)])
    wid = s * 2 + c
    pltpu.sync_copy(dst2_hbm.at[pl.ds(wid * CPW, CPW)], dst_v)
    plsc.subcore_barrier()

    def issue_src(j, sbuf, sem):
        pltpu.async_copy(src2_hbm.at[pl.ds(wid * CPW + j, 1)], sbuf, sem)

    def wait_src(j, sbuf, sem):
        pltpu.make_async_copy(src2_hbm.at[pl.ds(wid * CPW + j, 1)], sbuf,
                              sem).wait()

    def issue_g(sbuf, rows, sem):
        pltpu.async_copy(g_hbm.at[sbuf.at[0]], rows, sem)

    def wait_g(sbuf, rows, sem):
        pltpu.make_async_copy(g_hbm.at[sbuf.at[0]], rows, sem).wait()

    def scat(j, rows):
        pltpu.sync_copy(rows, acc_sh.at[dst_v.at[j]], add=True)

    n_pair = CPW // 2
    issue_src(0, s0, ssem0)
    issue_src(1, s1, ssem1)
    wait_src(0, s0, ssem0)
    issue_g(s0, rows0, gsem0)

    def body(k, carry):
        j0 = 2 * k
        wait_g(s0, rows0, gsem0)
        wait_src(j0 + 1, s1, ssem1)
        issue_g(s1, rows1, gsem1)
        scat(j0, rows0)
        issue_src(j0 + 2, s0, ssem0)
        wait_g(s1, rows1, gsem1)
        wait_src(j0 + 2, s0, ssem0)
        issue_g(s0, rows0, gsem0)
        scat(j0 + 1, rows1)
        issue_src(j0 + 3, s1, ssem1)
        return carry

    lax.fori_loop(0, n_pair - 1, body, 0)
    jl = 2 * (n_pair - 1)
    wait_g(s0, rows0, gsem0)
    wait_src(jl + 1, s1, ssem1)
    issue_g(s1, rows1, gsem1)
    scat(jl, rows0)
    wait_g(s1, rows1, gsem1)
    scat(jl + 1, rows1)
    plsc.subcore_barrier()
    pltpu.sync_copy(acc_sh.at[pl.ds(s * ROWS_PER_TILE, ROWS_PER_TILE)],
                    acc_out.at[c, pl.ds(s * ROWS_PER_TILE, ROWS_PER_TILE)])


@functools.partial(
    pl.kernel,
    out_type=jax.ShapeDtypeStruct((B_SEL, D), jnp.float32),
    mesh=_sc_mesh,
    scratch_types=[
        pltpu.VMEM((SEL_PER_W,), jnp.int32),
        pltpu.VMEM((SEL_PER_W, D), jnp.float32),
        pltpu.SemaphoreType.DMA,
    ],
)
def _gather_kernel(h_hbm, idx_hbm, out_hbm, idx_v, rows_v, sem):
    c = lax.axis_index("c")
    s = lax.axis_index("s")
    wid = s * 2 + c
    base = wid * SEL_PER_W
    pltpu.sync_copy(idx_hbm.at[pl.ds(base, SEL_PER_W)], idx_v)
    pltpu.async_copy(h_hbm.at[idx_v], rows_v, sem).wait()
    pltpu.sync_copy(rows_v, out_hbm.at[pl.ds(base, SEL_PER_W)])


_BLK = 1000


def _tc1_body(x_ref, w_ref, da_ref, db_ref, g_ref, dinv_ref):
    deg = da_ref[...] + db_ref[...] + 1.0
    dinv = lax.rsqrt(deg)
    dinv_ref[...] = dinv
    hw = jnp.dot(x_ref[...], w_ref[...], preferred_element_type=jnp.float32,
                 precision=lax.Precision.HIGHEST)
    g_ref[...] = hw * dinv


def _tc1(x, w1, deg_a, deg_b):
    n_blk = N_NODES // _BLK
    return pl.pallas_call(
        _tc1_body,
        grid=(n_blk,),
        in_specs=[
            pl.BlockSpec((_BLK, D), lambda i: (i, 0)),
            pl.BlockSpec((D, D), lambda i: (0, 0)),
            pl.BlockSpec((_BLK, 1), lambda i: (i, 0)),
            pl.BlockSpec((_BLK, 1), lambda i: (i, 0)),
        ],
        out_specs=[
            pl.BlockSpec((_BLK, D), lambda i: (i, 0)),
            pl.BlockSpec((_BLK, 1), lambda i: (i, 0)),
        ],
        out_shape=[
            jax.ShapeDtypeStruct((N_NODES, D), jnp.float32),
            jax.ShapeDtypeStruct((N_NODES, 1), jnp.float32),
        ],
    )(x, w1, deg_a, deg_b)


def _tc2_body(aa_ref, ab_ref, g_ref, dinv_ref, b_ref, w_ref, g2_ref):
    dinv = dinv_ref[...]
    pre = dinv * (aa_ref[...] + ab_ref[...] + g_ref[...]) + b_ref[...]
    h = jnp.maximum(pre, 0.0)
    hw = jnp.dot(h, w_ref[...], preferred_element_type=jnp.float32,
                 precision=lax.Precision.HIGHEST)
    g2_ref[...] = hw * dinv


def _tc2(acc_a, acc_b, g1, dinv, b1, w2):
    n_blk = N_NODES // _BLK
    return pl.pallas_call(
        _tc2_body,
        grid=(n_blk,),
        in_specs=[
            pl.BlockSpec((_BLK, D), lambda i: (i, 0)),
            pl.BlockSpec((_BLK, D), lambda i: (i, 0)),
            pl.BlockSpec((_BLK, D), lambda i: (i, 0)),
            pl.BlockSpec((_BLK, 1), lambda i: (i, 0)),
            pl.BlockSpec((1, D), lambda i: (0, 0)),
            pl.BlockSpec((D, D), lambda i: (0, 0)),
        ],
        out_specs=pl.BlockSpec((_BLK, D), lambda i: (i, 0)),
        out_shape=jax.ShapeDtypeStruct((N_NODES, D), jnp.float32),
    )(acc_a, acc_b, g1, dinv, b1, w2)


def _tc3_body(aa_ref, ab_ref, g_ref, dinv_ref, b_ref, h_ref):
    pre = dinv_ref[...] * (aa_ref[...] + ab_ref[...] + g_ref[...]) + b_ref[...]
    h_ref[...] = jnp.maximum(pre, 0.0)


def _tc3(acc_a, acc_b, g2, dinv, b2):
    n_blk = N_NODES // _BLK
    return pl.pallas_call(
        _tc3_body,
        grid=(n_blk,),
        in_specs=[
            pl.BlockSpec((_BLK, D), lambda i: (i, 0)),
            pl.BlockSpec((_BLK, D), lambda i: (i, 0)),
            pl.BlockSpec((_BLK, D), lambda i: (i, 0)),
            pl.BlockSpec((_BLK, 1), lambda i: (i, 0)),
            pl.BlockSpec((1, D), lambda i: (0, 0)),
        ],
        out_specs=pl.BlockSpec((_BLK, D), lambda i: (i, 0)),
        out_shape=jax.ShapeDtypeStruct((N_NODES, D), jnp.float32),
    )(acc_a, acc_b, g2, dinv, b2)


def _tc4_body(sel_ref, wt_ref, mut_ref, a1_ref, a2_ref, a3_ref, bh1_ref,
              w2_ref, bh2_ref, w3t_ref, bh3_ref, out_ref):
    z = (jnp.dot(sel_ref[...], a1_ref[...], preferred_element_type=jnp.float32,
                 precision=lax.Precision.HIGHEST)
         + jnp.dot(wt_ref[...], a2_ref[...], preferred_element_type=jnp.float32,
                   precision=lax.Precision.HIGHEST)
         + jnp.dot(mut_ref[...], a3_ref[...], preferred_element_type=jnp.float32,
                   precision=lax.Precision.HIGHEST)
         + bh1_ref[...])
    z = jnp.maximum(z, 0.0)
    z = jnp.dot(z, w2_ref[...], preferred_element_type=jnp.float32,
                precision=lax.Precision.HIGHEST) + bh2_ref[...]
    z = jnp.maximum(z, 0.0)
    o = jnp.sum(z * w3t_ref[...], axis=1, keepdims=True) + bh3_ref[...]
    out_ref[...] = o


def _tc4(sel, wt, mut, a1, a2, a3, bh1, w2, bh2, w3t, bh3):
    return pl.pallas_call(
        _tc4_body,
        out_shape=jax.ShapeDtypeStruct((B_SEL, 1), jnp.float32),
    )(sel, wt, mut, a1, a2, a3, bh1, w2, bh2, w3t, bh3)


def kernel(x, edge_index, var_node_idx, wt_onehot, mut_onehot,
           W1, b1, W2, b2, Wh1, bh1, Wh2, bh2, Wh3, bh3):
    zeros_acc = jnp.zeros((N_PAD, D), jnp.float32)
    ones128 = jnp.ones((CHUNK, D), jnp.float32)
    n_extra = E_PAD - E_EDGES
    src2 = jnp.concatenate(
        [edge_index[0], jnp.zeros((n_extra,), jnp.int32)]).reshape(-1, CHUNK)
    dst_fill = N_NODES + (jnp.arange(n_extra, dtype=jnp.int32)
                          % (N_PAD - N_NODES))
    dst2 = jnp.concatenate([edge_index[1], dst_fill]).reshape(-1, CHUNK)
    degp = _deg_kernel(dst2, ones128, zeros_acc)
    deg_a = degp[0, :N_NODES, 0:1]
    deg_b = degp[1, :N_NODES, 0:1]

    g1, dinv = _tc1(x, W1, deg_a, deg_b)

    acc1 = _seg_kernel(g1, src2, dst2, zeros_acc)
    g2 = _tc2(acc1[0, :N_NODES], acc1[1, :N_NODES], g1, dinv,
              b1.reshape(1, D), W2)

    acc2 = _seg_kernel(g2, src2, dst2, zeros_acc)
    h2 = _tc3(acc2[0, :N_NODES], acc2[1, :N_NODES], g2, dinv,
              b2.reshape(1, D))

    sel = _gather_kernel(h2, var_node_idx)

    out = _tc4(sel, wt_onehot, mut_onehot,
               Wh1[:D], Wh1[D:D + 20], Wh1[D + 20:D + 40],
               bh1.reshape(1, -1), Wh2, bh2.reshape(1, -1),
               Wh3.reshape(1, -1), bh3.reshape(1, 1))
    return out[:, 0]


# speedup vs baseline: 1.0002x; 1.0002x over previous
import functools

import jax
import jax.numpy as jnp
from jax import lax
from jax.experimental import pallas as pl
from jax.experimental.pallas import tpu as pltpu
from jax.experimental.pallas import tpu_sc as plsc

N_NODES = 10000
N_PAD = 10112
E_EDGES = 320000
D = 128
NW = 32
CHUNK = 128
CPW = 80
E_PAD = NW * CPW * CHUNK
ROWS_PER_TILE = N_PAD // 16
B_SEL = 4096
SEL_PER_W = B_SEL // NW

_sc_mesh = plsc.VectorSubcoreMesh(core_axis_name="c", subcore_axis_name="s")


@functools.partial(
    pl.kernel,
    out_type=jax.ShapeDtypeStruct((2, N_PAD, D), jnp.float32),
    mesh=_sc_mesh,
    scratch_types=[
        pltpu.VMEM((CPW, CHUNK), jnp.int32),
        pltpu.VMEM((CHUNK, D), jnp.float32),
        pltpu.VMEM_SHARED((N_PAD, D), jnp.float32),
    ],
)
def _deg_kernel(dst2_hbm, ones_hbm, zeros_hbm, deg_out, dst_v, ones_v, acc_sh):
    c = lax.axis_index("c")
    s = lax.axis_index("s")
    pltpu.sync_copy(zeros_hbm.at[pl.ds(s * ROWS_PER_TILE, ROWS_PER_TILE)],
                    acc_sh.at[pl.ds(s * ROWS_PER_TILE, ROWS_PER_TILE)])
    pltpu.sync_copy(ones_hbm, ones_v)
    wid = c * 16 + s
    pltpu.sync_copy(dst2_hbm.at[pl.ds(wid * CPW, CPW)], dst_v)
    plsc.subcore_barrier()

    def body(j, carry):
        pltpu.sync_copy(ones_v, acc_sh.at[dst_v.at[j]], add=True)
        return carry

    lax.fori_loop(0, CPW, body, 0)
    plsc.subcore_barrier()
    pltpu.sync_copy(acc_sh.at[pl.ds(s * ROWS_PER_TILE, ROWS_PER_TILE)],
                    deg_out.at[c, pl.ds(s * ROWS_PER_TILE, ROWS_PER_TILE)])


@functools.partial(
    pl.kernel,
    out_type=jax.ShapeDtypeStruct((2, N_PAD, D), jnp.float32),
    mesh=_sc_mesh,
    scratch_types=[
        pltpu.VMEM((CPW, CHUNK), jnp.int32),
        pltpu.VMEM((1, CHUNK), jnp.int32),
        pltpu.VMEM((1, CHUNK), jnp.int32),
        pltpu.VMEM((CHUNK, D), jnp.float32),
        pltpu.VMEM((CHUNK, D), jnp.float32),
        pltpu.SemaphoreType.DMA,
        pltpu.SemaphoreType.DMA,
        pltpu.SemaphoreType.DMA,
        pltpu.SemaphoreType.DMA,
        pltpu.VMEM_SHARED((N_PAD, D), jnp.float32),
    ],
)
def _seg_kernel(g_hbm, src2_hbm, dst2_hbm, zeros_hbm, acc_out, dst_v, s0, s1,
                rows0, rows1, gsem0, gsem1, ssem0, ssem1, acc_sh):
    c = lax.axis_index("c")
    s = lax.axis_index("s")
    pltpu.sync_copy(zeros_hbm.at[pl.ds(s * ROWS_PER_TILE, ROWS_PER_TILE)],
                    acc_sh.at[pl.ds(s * ROWS_PER_TILE, ROWS_PER_TILE)])
    wid = c * 16 + s
    pltpu.sync_copy(dst2_hbm.at[pl.ds(wid * CPW, CPW)], dst_v)
    plsc.subcore_barrier()

    def issue_src(j, sbuf, sem):
        pltpu.async_copy(src2_hbm.at[pl.ds(wid * CPW + j, 1)], sbuf, sem)

    def wait_src(j, sbuf, sem):
        pltpu.make_async_copy(src2_hbm.at[pl.ds(wid * CPW + j, 1)], sbuf,
                              sem).wait()

    def issue_g(sbuf, rows, sem):
        pltpu.async_copy(g_hbm.at[sbuf.at[0]], rows, sem)

    def wait_g(sbuf, rows, sem):
        pltpu.make_async_copy(g_hbm.at[sbuf.at[0]], rows, sem).wait()

    def scat(j, rows):
        pltpu.sync_copy(rows, acc_sh.at[dst_v.at[j]], add=True)

    n_pair = CPW // 2
    issue_src(0, s0, ssem0)
    issue_src(1, s1, ssem1)
    wait_src(0, s0, ssem0)
    issue_g(s0, rows0, gsem0)

    def body(k, carry):
        j0 = 2 * k
        wait_g(s0, rows0, gsem0)
        wait_src(j0 + 1, s1, ssem1)
        issue_g(s1, rows1, gsem1)
        scat(j0, rows0)
        issue_src(j0 + 2, s0, ssem0)
        wait_g(s1, rows1, gsem1)
        wait_src(j0 + 2, s0, ssem0)
        issue_g(s0, rows0, gsem0)
        scat(j0 + 1, rows1)
        issue_src(j0 + 3, s1, ssem1)
        return carry

    lax.fori_loop(0, n_pair - 1, body, 0)
    jl = 2 * (n_pair - 1)
    wait_g(s0, rows0, gsem0)
    wait_src(jl + 1, s1, ssem1)
    issue_g(s1, rows1, gsem1)
    scat(jl, rows0)
    wait_g(s1, rows1, gsem1)
    scat(jl + 1, rows1)
    plsc.subcore_barrier()
    pltpu.sync_copy(acc_sh.at[pl.ds(s * ROWS_PER_TILE, ROWS_PER_TILE)],
                    acc_out.at[c, pl.ds(s * ROWS_PER_TILE, ROWS_PER_TILE)])


@functools.partial(
    pl.kernel,
    out_type=jax.ShapeDtypeStruct((B_SEL, D), jnp.float32),
    mesh=_sc_mesh,
    scratch_types=[
        pltpu.VMEM((SEL_PER_W,), jnp.int32),
        pltpu.VMEM((SEL_PER_W, D), jnp.float32),
        pltpu.SemaphoreType.DMA,
    ],
)
def _gather_kernel(h_hbm, idx_hbm, out_hbm, idx_v, rows_v, sem):
    c = lax.axis_index("c")
    s = lax.axis_index("s")
    wid = c * 16 + s
    base = wid * SEL_PER_W
    pltpu.sync_copy(idx_hbm.at[pl.ds(base, SEL_PER_W)], idx_v)
    pltpu.async_copy(h_hbm.at[idx_v], rows_v, sem).wait()
    pltpu.sync_copy(rows_v, out_hbm.at[pl.ds(base, SEL_PER_W)])


_BLK = 1000


def _tc1_body(x_ref, w_ref, da_ref, db_ref, g_ref, dinv_ref):
    deg = da_ref[...] + db_ref[...] + 1.0
    dinv = lax.rsqrt(deg)
    dinv_ref[...] = dinv
    hw = jnp.dot(x_ref[...], w_ref[...], preferred_element_type=jnp.float32,
                 precision=lax.Precision.HIGHEST)
    g_ref[...] = hw * dinv


def _tc1(x, w1, deg_a, deg_b):
    n_blk = N_NODES // _BLK
    return pl.pallas_call(
        _tc1_body,
        grid=(n_blk,),
        in_specs=[
            pl.BlockSpec((_BLK, D), lambda i: (i, 0)),
            pl.BlockSpec((D, D), lambda i: (0, 0)),
            pl.BlockSpec((_BLK, 1), lambda i: (i, 0)),
            pl.BlockSpec((_BLK, 1), lambda i: (i, 0)),
        ],
        out_specs=[
            pl.BlockSpec((_BLK, D), lambda i: (i, 0)),
            pl.BlockSpec((_BLK, 1), lambda i: (i, 0)),
        ],
        out_shape=[
            jax.ShapeDtypeStruct((N_NODES, D), jnp.float32),
            jax.ShapeDtypeStruct((N_NODES, 1), jnp.float32),
        ],
    )(x, w1, deg_a, deg_b)


def _tc2_body(aa_ref, ab_ref, g_ref, dinv_ref, b_ref, w_ref, g2_ref):
    dinv = dinv_ref[...]
    pre = dinv * (aa_ref[...] + ab_ref[...] + g_ref[...]) + b_ref[...]
    h = jnp.maximum(pre, 0.0)
    hw = jnp.dot(h, w_ref[...], preferred_element_type=jnp.float32,
                 precision=lax.Precision.HIGHEST)
    g2_ref[...] = hw * dinv


def _tc2(acc_a, acc_b, g1, dinv, b1, w2):
    n_blk = N_NODES // _BLK
    return pl.pallas_call(
        _tc2_body,
        grid=(n_blk,),
        in_specs=[
            pl.BlockSpec((_BLK, D), lambda i: (i, 0)),
            pl.BlockSpec((_BLK, D), lambda i: (i, 0)),
            pl.BlockSpec((_BLK, D), lambda i: (i, 0)),
            pl.BlockSpec((_BLK, 1), lambda i: (i, 0)),
            pl.BlockSpec((1, D), lambda i: (0, 0)),
            pl.BlockSpec((D, D), lambda i: (0, 0)),
        ],
        out_specs=pl.BlockSpec((_BLK, D), lambda i: (i, 0)),
        out_shape=jax.ShapeDtypeStruct((N_NODES, D), jnp.float32),
    )(acc_a, acc_b, g1, dinv, b1, w2)


def _tc3_body(aa_ref, ab_ref, g_ref, dinv_ref, b_ref, h_ref):
    pre = dinv_ref[...] * (aa_ref[...] + ab_ref[...] + g_ref[...]) + b_ref[...]
    h_ref[...] = jnp.maximum(pre, 0.0)


def _tc3(acc_a, acc_b, g2, dinv, b2):
    n_blk = N_NODES // _BLK
    return pl.pallas_call(
        _tc3_body,
        grid=(n_blk,),
        in_specs=[
            pl.BlockSpec((_BLK, D), lambda i: (i, 0)),
            pl.BlockSpec((_BLK, D), lambda i: (i, 0)),
            pl.BlockSpec((_BLK, D), lambda i: (i, 0)),
            pl.BlockSpec((_BLK, 1), lambda i: (i, 0)),
            pl.BlockSpec((1, D), lambda i: (0, 0)),
        ],
        out_specs=pl.BlockSpec((_BLK, D), lambda i: (i, 0)),
        out_shape=jax.ShapeDtypeStruct((N_NODES, D), jnp.float32),
    )(acc_a, acc_b, g2, dinv, b2)


def _tc4_body(sel_ref, wt_ref, mut_ref, a1_ref, a2_ref, a3_ref, bh1_ref,
              w2_ref, bh2_ref, w3t_ref, bh3_ref, out_ref):
    z = (jnp.dot(sel_ref[...], a1_ref[...], preferred_element_type=jnp.float32,
                 precision=lax.Precision.HIGHEST)
         + jnp.dot(wt_ref[...], a2_ref[...], preferred_element_type=jnp.float32,
                   precision=lax.Precision.HIGHEST)
         + jnp.dot(mut_ref[...], a3_ref[...], preferred_element_type=jnp.float32,
                   precision=lax.Precision.HIGHEST)
         + bh1_ref[...])
    z = jnp.maximum(z, 0.0)
    z = jnp.dot(z, w2_ref[...], preferred_element_type=jnp.float32,
                precision=lax.Precision.HIGHEST) + bh2_ref[...]
    z = jnp.maximum(z, 0.0)
    o = jnp.sum(z * w3t_ref[...], axis=1, keepdims=True) + bh3_ref[...]
    out_ref[...] = o


def _tc4(sel, wt, mut, a1, a2, a3, bh1, w2, bh2, w3t, bh3):
    return pl.pallas_call(
        _tc4_body,
        out_shape=jax.ShapeDtypeStruct((B_SEL, 1), jnp.float32),
    )(sel, wt, mut, a1, a2, a3, bh1, w2, bh2, w3t, bh3)


def kernel(x, edge_index, var_node_idx, wt_onehot, mut_onehot,
           W1, b1, W2, b2, Wh1, bh1, Wh2, bh2, Wh3, bh3):
    zeros_acc = jnp.zeros((N_PAD, D), jnp.float32)
    ones128 = jnp.ones((CHUNK, D), jnp.float32)
    n_extra = E_PAD - E_EDGES
    src2 = jnp.concatenate(
        [edge_index[0], jnp.zeros((n_extra,), jnp.int32)]).reshape(-1, CHUNK)
    dst_fill = N_NODES + (jnp.arange(n_extra, dtype=jnp.int32)
                          % (N_PAD - N_NODES))
    dst2 = jnp.concatenate([edge_index[1], dst_fill]).reshape(-1, CHUNK)
    degp = _deg_kernel(dst2, ones128, zeros_acc)
    deg_a = degp[0, :N_NODES, 0:1]
    deg_b = degp[1, :N_NODES, 0:1]

    g1, dinv = _tc1(x, W1, deg_a, deg_b)

    acc1 = _seg_kernel(g1, src2, dst2, zeros_acc)
    g2 = _tc2(acc1[0, :N_NODES], acc1[1, :N_NODES], g1, dinv,
              b1.reshape(1, D), W2)

    acc2 = _seg_kernel(g2, src2, dst2, zeros_acc)
    h2 = _tc3(acc2[0, :N_NODES], acc2[1, :N_NODES], g2, dinv,
              b2.reshape(1, D))

    sel = _gather_kernel(h2, var_node_idx)

    out = _tc4(sel, wt_onehot, mut_onehot,
               Wh1[:D], Wh1[D:D + 20], Wh1[D + 20:D + 40],
               bh1.reshape(1, -1), Wh2, bh2.reshape(1, -1),
               Wh3.reshape(1, -1), bh3.reshape(1, 1))
    return out[:, 0]


# speedup vs baseline: 2.6880x; 2.6875x over previous
import functools

import jax
import jax.numpy as jnp
from jax import lax
from jax.experimental import pallas as pl
from jax.experimental.pallas import tpu as pltpu
from jax.experimental.pallas import tpu_sc as plsc

N_NODES = 10000
N_PAD = 10112
E_EDGES = 320000
D = 128
NW = 32
CHUNK = 128
CPW = 80
E_PAD = NW * CPW * CHUNK
ROWS_PER_TILE = N_PAD // 16
B_SEL = 4096
SEL_PER_W = B_SEL // NW

_sc_mesh = plsc.VectorSubcoreMesh(core_axis_name="c", subcore_axis_name="s")


@functools.partial(
    pl.kernel,
    out_type=jax.ShapeDtypeStruct((2, N_PAD, D), jnp.float32),
    mesh=_sc_mesh,
    scratch_types=[
        pltpu.VMEM((CPW, CHUNK), jnp.int32),
        pltpu.VMEM((CHUNK, D), jnp.float32),
        pltpu.VMEM_SHARED((N_PAD, D), jnp.float32),
    ],
)
def _deg_kernel(dst2_hbm, ones_hbm, zeros_hbm, deg_out, dst_v, ones_v, acc_sh):
    c = lax.axis_index("c")
    s = lax.axis_index("s")
    pltpu.sync_copy(zeros_hbm.at[pl.ds(s * ROWS_PER_TILE, ROWS_PER_TILE)],
                    acc_sh.at[pl.ds(s * ROWS_PER_TILE, ROWS_PER_TILE)])
    pltpu.sync_copy(ones_hbm, ones_v)
    wid = c * 16 + s
    pltpu.sync_copy(dst2_hbm.at[pl.ds(wid * CPW, CPW)], dst_v)
    plsc.subcore_barrier()

    def body(j, carry):
        pltpu.sync_copy(ones_v, acc_sh.at[dst_v.at[j]], add=True)
        return carry

    lax.fori_loop(0, CPW, body, 0)
    plsc.subcore_barrier()
    pltpu.sync_copy(acc_sh.at[pl.ds(s * ROWS_PER_TILE, ROWS_PER_TILE)],
                    deg_out.at[c, pl.ds(s * ROWS_PER_TILE, ROWS_PER_TILE)])


@functools.partial(
    pl.kernel,
    out_type=jax.ShapeDtypeStruct((2, N_PAD, D), jnp.float32),
    mesh=_sc_mesh,
    scratch_types=[
        pltpu.VMEM((CPW, CHUNK), jnp.int32),
        pltpu.VMEM((1, CHUNK), jnp.int32),
        pltpu.VMEM((1, CHUNK), jnp.int32),
        pltpu.VMEM((CHUNK, D), jnp.float32),
        pltpu.VMEM((CHUNK, D), jnp.float32),
        pltpu.SemaphoreType.DMA,
        pltpu.SemaphoreType.DMA,
        pltpu.SemaphoreType.DMA,
        pltpu.SemaphoreType.DMA,
        pltpu.VMEM_SHARED((N_PAD, D), jnp.float32),
    ],
)
def _seg_kernel(g_hbm, src2_hbm, dst2_hbm, zeros_hbm, acc_out, dst_v, s0, s1,
                rows0, rows1, gsem0, gsem1, ssem0, ssem1, acc_sh):
    c = lax.axis_index("c")
    s = lax.axis_index("s")
    pltpu.sync_copy(zeros_hbm.at[pl.ds(s * ROWS_PER_TILE, ROWS_PER_TILE)],
                    acc_sh.at[pl.ds(s * ROWS_PER_TILE, ROWS_PER_TILE)])
    wid = c * 16 + s
    pltpu.sync_copy(dst2_hbm.at[pl.ds(wid * CPW, CPW)], dst_v)
    plsc.subcore_barrier()

    def issue_src(j, sbuf, sem):
        pltpu.async_copy(src2_hbm.at[pl.ds(wid * CPW + j, 1)], sbuf, sem)

    def wait_src(j, sbuf, sem):
        pltpu.make_async_copy(src2_hbm.at[pl.ds(wid * CPW + j, 1)], sbuf,
                              sem).wait()

    def issue_g(sbuf, rows, sem):
        pltpu.async_copy(g_hbm.at[sbuf.at[0]], rows, sem)

    def wait_g(sbuf, rows, sem):
        pltpu.make_async_copy(g_hbm.at[sbuf.at[0]], rows, sem).wait()

    def scat(j, rows):
        pltpu.sync_copy(rows, acc_sh.at[dst_v.at[j]], add=True)

    n_pair = CPW // 2
    issue_src(0, s0, ssem0)
    issue_src(1, s1, ssem1)
    wait_src(0, s0, ssem0)
    issue_g(s0, rows0, gsem0)

    def body(k, carry):
        j0 = 2 * k
        wait_g(s0, rows0, gsem0)
        wait_src(j0 + 1, s1, ssem1)
        issue_g(s1, rows1, gsem1)
        scat(j0, rows0)
        issue_src(j0 + 2, s0, ssem0)
        wait_g(s1, rows1, gsem1)
        wait_src(j0 + 2, s0, ssem0)
        issue_g(s0, rows0, gsem0)
        scat(j0 + 1, rows1)
        issue_src(j0 + 3, s1, ssem1)
        return carry

    lax.fori_loop(0, n_pair - 1, body, 0)
    jl = 2 * (n_pair - 1)
    wait_g(s0, rows0, gsem0)
    wait_src(jl + 1, s1, ssem1)
    issue_g(s1, rows1, gsem1)
    scat(jl, rows0)
    wait_g(s1, rows1, gsem1)
    scat(jl + 1, rows1)
    plsc.subcore_barrier()
    pltpu.sync_copy(acc_sh.at[pl.ds(s * ROWS_PER_TILE, ROWS_PER_TILE)],
                    acc_out.at[c, pl.ds(s * ROWS_PER_TILE, ROWS_PER_TILE)])


@functools.partial(
    pl.kernel,
    out_type=jax.ShapeDtypeStruct((B_SEL, D), jnp.float32),
    mesh=_sc_mesh,
    scratch_types=[
        pltpu.VMEM((SEL_PER_W,), jnp.int32),
        pltpu.VMEM((SEL_PER_W, D), jnp.float32),
        pltpu.SemaphoreType.DMA,
    ],
)
def _gather_kernel(h_hbm, idx_hbm, out_hbm, idx_v, rows_v, sem):
    c = lax.axis_index("c")
    s = lax.axis_index("s")
    wid = c * 16 + s
    base = wid * SEL_PER_W
    pltpu.sync_copy(idx_hbm.at[pl.ds(base, SEL_PER_W)], idx_v)
    pltpu.async_copy(h_hbm.at[idx_v], rows_v, sem).wait()
    pltpu.sync_copy(rows_v, out_hbm.at[pl.ds(base, SEL_PER_W)])


_BLK = 1000


def _tc1_body(x_ref, w_ref, da_ref, db_ref, g_ref, dinv_ref):
    deg = da_ref[...] + db_ref[...] + 1.0
    dinv = lax.rsqrt(deg)
    dinv_ref[...] = dinv
    hw = jnp.dot(x_ref[...], w_ref[...], preferred_element_type=jnp.float32,
                 precision=lax.Precision.HIGHEST)
    g_ref[...] = hw * dinv


def _tc1(x, w1, deg_a, deg_b):
    n_blk = N_NODES // _BLK
    return pl.pallas_call(
        _tc1_body,
        grid=(n_blk,),
        in_specs=[
            pl.BlockSpec((_BLK, D), lambda i: (i, 0)),
            pl.BlockSpec((D, D), lambda i: (0, 0)),
            pl.BlockSpec((_BLK, 1), lambda i: (i, 0)),
            pl.BlockSpec((_BLK, 1), lambda i: (i, 0)),
        ],
        out_specs=[
            pl.BlockSpec((_BLK, D), lambda i: (i, 0)),
            pl.BlockSpec((_BLK, 1), lambda i: (i, 0)),
        ],
        out_shape=[
            jax.ShapeDtypeStruct((N_NODES, D), jnp.float32),
            jax.ShapeDtypeStruct((N_NODES, 1), jnp.float32),
        ],
    )(x, w1, deg_a, deg_b)


def _tc2_body(aa_ref, ab_ref, g_ref, dinv_ref, b_ref, w_ref, g2_ref):
    dinv = dinv_ref[...]
    pre = dinv * (aa_ref[...] + ab_ref[...] + g_ref[...]) + b_ref[...]
    h = jnp.maximum(pre, 0.0)
    hw = jnp.dot(h, w_ref[...], preferred_element_type=jnp.float32,
                 precision=lax.Precision.HIGHEST)
    g2_ref[...] = hw * dinv


def _tc2(acc_a, acc_b, g1, dinv, b1, w2):
    n_blk = N_NODES // _BLK
    return pl.pallas_call(
        _tc2_body,
        grid=(n_blk,),
        in_specs=[
            pl.BlockSpec((_BLK, D), lambda i: (i, 0)),
            pl.BlockSpec((_BLK, D), lambda i: (i, 0)),
            pl.BlockSpec((_BLK, D), lambda i: (i, 0)),
            pl.BlockSpec((_BLK, 1), lambda i: (i, 0)),
            pl.BlockSpec((1, D), lambda i: (0, 0)),
            pl.BlockSpec((D, D), lambda i: (0, 0)),
        ],
        out_specs=pl.BlockSpec((_BLK, D), lambda i: (i, 0)),
        out_shape=jax.ShapeDtypeStruct((N_NODES, D), jnp.float32),
    )(acc_a, acc_b, g1, dinv, b1, w2)


def _tc3_body(aa_ref, ab_ref, g_ref, dinv_ref, b_ref, h_ref):
    pre = dinv_ref[...] * (aa_ref[...] + ab_ref[...] + g_ref[...]) + b_ref[...]
    h_ref[...] = jnp.maximum(pre, 0.0)


def _tc3(acc_a, acc_b, g2, dinv, b2):
    n_blk = N_NODES // _BLK
    return pl.pallas_call(
        _tc3_body,
        grid=(n_blk,),
        in_specs=[
            pl.BlockSpec((_BLK, D), lambda i: (i, 0)),
            pl.BlockSpec((_BLK, D), lambda i: (i, 0)),
            pl.BlockSpec((_BLK, D), lambda i: (i, 0)),
            pl.BlockSpec((_BLK, 1), lambda i: (i, 0)),
            pl.BlockSpec((1, D), lambda i: (0, 0)),
        ],
        out_specs=pl.BlockSpec((_BLK, D), lambda i: (i, 0)),
        out_shape=jax.ShapeDtypeStruct((N_NODES, D), jnp.float32),
    )(acc_a, acc_b, g2, dinv, b2)


def _tc4_body(sel_ref, wt_ref, mut_ref, a1_ref, a2_ref, a3_ref, bh1_ref,
              w2_ref, bh2_ref, w3t_ref, bh3_ref, out_ref):
    z = (jnp.dot(sel_ref[...], a1_ref[...], preferred_element_type=jnp.float32,
                 precision=lax.Precision.HIGHEST)
         + jnp.dot(wt_ref[...], a2_ref[...], preferred_element_type=jnp.float32,
                   precision=lax.Precision.HIGHEST)
         + jnp.dot(mut_ref[...], a3_ref[...], preferred_element_type=jnp.float32,
                   precision=lax.Precision.HIGHEST)
         + bh1_ref[...])
    z = jnp.maximum(z, 0.0)
    z = jnp.dot(z, w2_ref[...], preferred_element_type=jnp.float32,
                precision=lax.Precision.HIGHEST) + bh2_ref[...]
    z = jnp.maximum(z, 0.0)
    o = jnp.sum(z * w3t_ref[...], axis=1, keepdims=True) + bh3_ref[...]
    out_ref[...] = o


def _tc4(sel, wt, mut, a1, a2, a3, bh1, w2, bh2, w3t, bh3):
    return pl.pallas_call(
        _tc4_body,
        out_shape=jax.ShapeDtypeStruct((B_SEL, 1), jnp.float32),
    )(sel, wt, mut, a1, a2, a3, bh1, w2, bh2, w3t, bh3)


def kernel(x, edge_index, var_node_idx, wt_onehot, mut_onehot,
           W1, b1, W2, b2, Wh1, bh1, Wh2, bh2, Wh3, bh3):
    zeros_acc = jnp.zeros((N_PAD, D), jnp.float32)
    ones128 = jnp.ones((CHUNK, D), jnp.float32)
    n_extra = E_PAD - E_EDGES
    ar = jnp.arange(n_extra, dtype=jnp.int32)
    src_fill = (ar * 997) % N_NODES
    src2 = jnp.concatenate([edge_index[0], src_fill]).reshape(-1, CHUNK)
    dst_fill = N_NODES + (ar * 13) % (N_PAD - N_NODES)
    dst2 = jnp.concatenate([edge_index[1], dst_fill]).reshape(-1, CHUNK)
    degp = _deg_kernel(dst2, ones128, zeros_acc)
    deg_a = degp[0, :N_NODES, 0:1]
    deg_b = degp[1, :N_NODES, 0:1]

    g1, dinv = _tc1(x, W1, deg_a, deg_b)

    acc1 = _seg_kernel(g1, src2, dst2, zeros_acc)
    g2 = _tc2(acc1[0, :N_NODES], acc1[1, :N_NODES], g1, dinv,
              b1.reshape(1, D), W2)

    acc2 = _seg_kernel(g2, src2, dst2, zeros_acc)
    h2 = _tc3(acc2[0, :N_NODES], acc2[1, :N_NODES], g2, dinv,
              b2.reshape(1, D))

    sel = _gather_kernel(h2, var_node_idx)

    out = _tc4(sel, wt_onehot, mut_onehot,
               Wh1[:D], Wh1[D:D + 20], Wh1[D + 20:D + 40],
               bh1.reshape(1, -1), Wh2, bh2.reshape(1, -1),
               Wh3.reshape(1, -1), bh3.reshape(1, 1))
    return out[:, 0]


# speedup vs baseline: 2.8793x; 1.0712x over previous
import functools

import jax
import jax.numpy as jnp
from jax import lax
from jax.experimental import pallas as pl
from jax.experimental.pallas import tpu as pltpu
from jax.experimental.pallas import tpu_sc as plsc

N_NODES = 10000
N_PAD = 10112
E_EDGES = 320000
D = 128
NW = 32
CHUNK = 128
CPW = 80
E_PAD = NW * CPW * CHUNK
ROWS_PER_TILE = N_PAD // 16
B_SEL = 4096
SEL_PER_W = B_SEL // NW

_sc_mesh = plsc.VectorSubcoreMesh(core_axis_name="c", subcore_axis_name="s")


@functools.partial(
    pl.kernel,
    out_type=jax.ShapeDtypeStruct((2, N_PAD, D), jnp.float32),
    mesh=_sc_mesh,
    scratch_types=[
        pltpu.VMEM((CPW, CHUNK), jnp.int32),
        pltpu.VMEM((CHUNK, D), jnp.float32),
        pltpu.VMEM_SHARED((N_PAD, D), jnp.float32),
    ],
)
def _deg_kernel(dst2_hbm, ones_hbm, zeros_hbm, deg_out, dst_v, ones_v, acc_sh):
    c = lax.axis_index("c")
    s = lax.axis_index("s")
    pltpu.sync_copy(zeros_hbm.at[pl.ds(s * ROWS_PER_TILE, ROWS_PER_TILE)],
                    acc_sh.at[pl.ds(s * ROWS_PER_TILE, ROWS_PER_TILE)])
    pltpu.sync_copy(ones_hbm, ones_v)
    wid = c * 16 + s
    pltpu.sync_copy(dst2_hbm.at[pl.ds(wid * CPW, CPW)], dst_v)
    plsc.subcore_barrier()

    def body(j, carry):
        pltpu.sync_copy(ones_v, acc_sh.at[dst_v.at[j]], add=True)
        return carry

    lax.fori_loop(0, CPW, body, 0)
    plsc.subcore_barrier()
    pltpu.sync_copy(acc_sh.at[pl.ds(s * ROWS_PER_TILE, ROWS_PER_TILE)],
                    deg_out.at[c, pl.ds(s * ROWS_PER_TILE, ROWS_PER_TILE)])


@functools.partial(
    pl.kernel,
    out_type=jax.ShapeDtypeStruct((2, N_PAD, D), jnp.float32),
    mesh=_sc_mesh,
    scratch_types=[
        pltpu.VMEM((CPW, CHUNK), jnp.int32),
        pltpu.VMEM((1, CHUNK), jnp.int32),
        pltpu.VMEM((1, CHUNK), jnp.int32),
        pltpu.VMEM((CHUNK, D), jnp.float32),
        pltpu.VMEM((CHUNK, D), jnp.float32),
        pltpu.SemaphoreType.DMA,
        pltpu.SemaphoreType.DMA,
        pltpu.SemaphoreType.DMA,
        pltpu.SemaphoreType.DMA,
        pltpu.VMEM_SHARED((N_PAD, D), jnp.float32),
    ],
)
def _seg_kernel(g_hbm, src2_hbm, dst2_hbm, zeros_hbm, acc_out, dst_v, s0, s1,
                rows0, rows1, gsem0, gsem1, ssem0, ssem1, acc_sh):
    c = lax.axis_index("c")
    s = lax.axis_index("s")
    pltpu.sync_copy(zeros_hbm.at[pl.ds(s * ROWS_PER_TILE, ROWS_PER_TILE)],
                    acc_sh.at[pl.ds(s * ROWS_PER_TILE, ROWS_PER_TILE)])
    wid = c * 16 + s
    pltpu.sync_copy(dst2_hbm.at[pl.ds(wid * CPW, CPW)], dst_v)
    plsc.subcore_barrier()

    def issue_src(j, sbuf, sem):
        pltpu.async_copy(src2_hbm.at[pl.ds(wid * CPW + j, 1)], sbuf, sem)

    def wait_src(j, sbuf, sem):
        pltpu.make_async_copy(src2_hbm.at[pl.ds(wid * CPW + j, 1)], sbuf,
                              sem).wait()

    def issue_g(sbuf, rows, sem):
        pltpu.async_copy(g_hbm.at[sbuf.at[0]], rows, sem)

    def wait_g(sbuf, rows, sem):
        pltpu.make_async_copy(g_hbm.at[sbuf.at[0]], rows, sem).wait()

    def scat(j, rows):
        pltpu.sync_copy(rows, acc_sh.at[dst_v.at[j]], add=True)

    n_pair = CPW // 2
    issue_src(0, s0, ssem0)
    issue_src(1, s1, ssem1)
    wait_src(0, s0, ssem0)
    issue_g(s0, rows0, gsem0)

    def body(k, carry):
        j0 = 2 * k
        wait_g(s0, rows0, gsem0)
        wait_src(j0 + 1, s1, ssem1)
        issue_g(s1, rows1, gsem1)
        scat(j0, rows0)
        issue_src(j0 + 2, s0, ssem0)
        wait_g(s1, rows1, gsem1)
        wait_src(j0 + 2, s0, ssem0)
        issue_g(s0, rows0, gsem0)
        scat(j0 + 1, rows1)
        issue_src(j0 + 3, s1, ssem1)
        return carry

    lax.fori_loop(0, n_pair - 1, body, 0)
    jl = 2 * (n_pair - 1)
    wait_g(s0, rows0, gsem0)
    wait_src(jl + 1, s1, ssem1)
    issue_g(s1, rows1, gsem1)
    scat(jl, rows0)
    wait_g(s1, rows1, gsem1)
    scat(jl + 1, rows1)
    plsc.subcore_barrier()
    pltpu.sync_copy(acc_sh.at[pl.ds(s * ROWS_PER_TILE, ROWS_PER_TILE)],
                    acc_out.at[c, pl.ds(s * ROWS_PER_TILE, ROWS_PER_TILE)])


@functools.partial(
    pl.kernel,
    out_type=jax.ShapeDtypeStruct((B_SEL, D), jnp.float32),
    mesh=_sc_mesh,
    scratch_types=[
        pltpu.VMEM((SEL_PER_W,), jnp.int32),
        pltpu.VMEM((SEL_PER_W, D), jnp.float32),
        pltpu.SemaphoreType.DMA,
    ],
)
def _gather_kernel(h_hbm, idx_hbm, out_hbm, idx_v, rows_v, sem):
    c = lax.axis_index("c")
    s = lax.axis_index("s")
    wid = c * 16 + s
    base = wid * SEL_PER_W
    pltpu.sync_copy(idx_hbm.at[pl.ds(base, SEL_PER_W)], idx_v)
    pltpu.async_copy(h_hbm.at[idx_v], rows_v, sem).wait()
    pltpu.sync_copy(rows_v, out_hbm.at[pl.ds(base, SEL_PER_W)])


_BLK = 1000


def _tc1_body(x_ref, w_ref, da_ref, db_ref, g_ref, dinv_ref):
    deg = da_ref[...] + db_ref[...] + 1.0
    dinv = lax.rsqrt(deg)
    dinv_ref[...] = dinv
    hw = jnp.dot(x_ref[...], w_ref[...], preferred_element_type=jnp.float32)
    g_ref[...] = hw * dinv


def _tc1(x, w1, deg_a, deg_b):
    n_blk = N_NODES // _BLK
    return pl.pallas_call(
        _tc1_body,
        grid=(n_blk,),
        in_specs=[
            pl.BlockSpec((_BLK, D), lambda i: (i, 0)),
            pl.BlockSpec((D, D), lambda i: (0, 0)),
            pl.BlockSpec((_BLK, 1), lambda i: (i, 0)),
            pl.BlockSpec((_BLK, 1), lambda i: (i, 0)),
        ],
        out_specs=[
            pl.BlockSpec((_BLK, D), lambda i: (i, 0)),
            pl.BlockSpec((_BLK, 1), lambda i: (i, 0)),
        ],
        out_shape=[
            jax.ShapeDtypeStruct((N_NODES, D), jnp.float32),
            jax.ShapeDtypeStruct((N_NODES, 1), jnp.float32),
        ],
    )(x, w1, deg_a, deg_b)


def _tc2_body(aa_ref, ab_ref, g_ref, dinv_ref, b_ref, w_ref, g2_ref):
    dinv = dinv_ref[...]
    pre = dinv * (aa_ref[0] + ab_ref[0] + g_ref[...]) + b_ref[...]
    h = jnp.maximum(pre, 0.0)
    hw = jnp.dot(h, w_ref[...], preferred_element_type=jnp.float32)
    g2_ref[...] = hw * dinv


def _tc2(acc, g1, dinv, b1, w2):
    n_blk = N_NODES // _BLK
    return pl.pallas_call(
        _tc2_body,
        grid=(n_blk,),
        in_specs=[
            pl.BlockSpec((1, _BLK, D), lambda i: (0, i, 0)),
            pl.BlockSpec((1, _BLK, D), lambda i: (1, i, 0)),
            pl.BlockSpec((_BLK, D), lambda i: (i, 0)),
            pl.BlockSpec((_BLK, 1), lambda i: (i, 0)),
            pl.BlockSpec((1, D), lambda i: (0, 0)),
            pl.BlockSpec((D, D), lambda i: (0, 0)),
        ],
        out_specs=pl.BlockSpec((_BLK, D), lambda i: (i, 0)),
        out_shape=jax.ShapeDtypeStruct((N_NODES, D), jnp.float32),
    )(acc, acc, g1, dinv, b1, w2)


def _tc3_body(aa_ref, ab_ref, g_ref, dinv_ref, b_ref, h_ref):
    pre = dinv_ref[...] * (aa_ref[0] + ab_ref[0] + g_ref[...]) + b_ref[...]
    h_ref[...] = jnp.maximum(pre, 0.0)


def _tc3(acc, g2, dinv, b2):
    n_blk = N_NODES // _BLK
    return pl.pallas_call(
        _tc3_body,
        grid=(n_blk,),
        in_specs=[
            pl.BlockSpec((1, _BLK, D), lambda i: (0, i, 0)),
            pl.BlockSpec((1, _BLK, D), lambda i: (1, i, 0)),
            pl.BlockSpec((_BLK, D), lambda i: (i, 0)),
            pl.BlockSpec((_BLK, 1), lambda i: (i, 0)),
            pl.BlockSpec((1, D), lambda i: (0, 0)),
        ],
        out_specs=pl.BlockSpec((_BLK, D), lambda i: (i, 0)),
        out_shape=jax.ShapeDtypeStruct((N_NODES, D), jnp.float32),
    )(acc, acc, g2, dinv, b2)


def _tc4_body(sel_ref, wt_ref, mut_ref, a1_ref, a2_ref, a3_ref, bh1_ref,
              w2_ref, bh2_ref, w3t_ref, bh3_ref, out_ref):
    z = (jnp.dot(sel_ref[...], a1_ref[...], preferred_element_type=jnp.float32)
         + jnp.dot(wt_ref[...], a2_ref[...], preferred_element_type=jnp.float32)
         + jnp.dot(mut_ref[...], a3_ref[...], preferred_element_type=jnp.float32)
         + bh1_ref[...])
    z = jnp.maximum(z, 0.0)
    z = jnp.dot(z, w2_ref[...], preferred_element_type=jnp.float32) + bh2_ref[...]
    z = jnp.maximum(z, 0.0)
    o = jnp.sum(z * w3t_ref[...], axis=1, keepdims=True) + bh3_ref[...]
    out_ref[...] = o


def _tc4(sel, wt, mut, a1, a2, a3, bh1, w2, bh2, w3t, bh3):
    return pl.pallas_call(
        _tc4_body,
        out_shape=jax.ShapeDtypeStruct((B_SEL, 1), jnp.float32),
    )(sel, wt, mut, a1, a2, a3, bh1, w2, bh2, w3t, bh3)


def kernel(x, edge_index, var_node_idx, wt_onehot, mut_onehot,
           W1, b1, W2, b2, Wh1, bh1, Wh2, bh2, Wh3, bh3):
    zeros_acc = jnp.zeros((N_PAD, D), jnp.float32)
    ones128 = jnp.ones((CHUNK, D), jnp.float32)
    n_extra = E_PAD - E_EDGES
    ar = jnp.arange(n_extra, dtype=jnp.int32)
    src_fill = (ar * 997) % N_NODES
    src2 = jnp.concatenate([edge_index[0], src_fill]).reshape(-1, CHUNK)
    dst_fill = N_NODES + (ar * 13) % (N_PAD - N_NODES)
    dst2 = jnp.concatenate([edge_index[1], dst_fill]).reshape(-1, CHUNK)
    degp = _deg_kernel(dst2, ones128, zeros_acc)
    deg_a = degp[0, :N_NODES, 0:1]
    deg_b = degp[1, :N_NODES, 0:1]

    g1, dinv = _tc1(x, W1, deg_a, deg_b)

    acc1 = _seg_kernel(g1, src2, dst2, zeros_acc)
    g2 = _tc2(acc1, g1, dinv, b1.reshape(1, D), W2)

    acc2 = _seg_kernel(g2, src2, dst2, zeros_acc)
    h2 = _tc3(acc2, g2, dinv, b2.reshape(1, D))

    sel = _gather_kernel(h2, var_node_idx)

    out = _tc4(sel, wt_onehot, mut_onehot,
               Wh1[:D], Wh1[D:D + 20], Wh1[D + 20:D + 40],
               bh1.reshape(1, -1), Wh2, bh2.reshape(1, -1),
               Wh3.reshape(1, -1), bh3.reshape(1, 1))
    return out[:, 0]
